# baseline (speedup 1.0000x reference)
import functools

import jax
import jax.numpy as jnp
from jax import lax
from jax.experimental import pallas as pl
from jax.experimental.pallas import tpu as pltpu
from jax.experimental.pallas import tpu_sc as plsc

_D = _H = _W = 256
_N = 1048576
_DHW = _D * _H * _W

_NC = 2
_NS = 16
_NW = _NC * _NS
_PER_W = _N // _NW
_CHUNK = 2048
_NCHUNK = _PER_W // _CHUNK

_ZY_OFFS = (0, _W, _H * _W, _H * _W + _W)


def _tec_body(c0, c1, c2, c3, c4, c5, c6, vol_hbm, out_hbm,
              cols_v, idx_v, w_v, val_v, out_v, sems):
    cols_hbm = (c0, c1, c2, c3, c4, c5, c6)
    wid = lax.axis_index("s") * _NC + lax.axis_index("c")
    base_w = wid * _PER_W

    def prep(c, b):
        base = base_w + c * _CHUNK
        for q in range(7):
            pltpu.sync_copy(cols_hbm[q].at[pl.ds(base, _CHUNK)],
                            cols_v[b * 7 + q])

        def compute_idx(j, carry):
            dsl = pl.ds(j * 16, 16)
            f = cols_v[b * 7 + 6][dsl]
            yin = f == 0.0
            x = jnp.where(yin, cols_v[b * 7 + 0][dsl], cols_v[b * 7 + 3][dsl])
            y = jnp.where(yin, cols_v[b * 7 + 1][dsl], cols_v[b * 7 + 4][dsl])
            z = jnp.where(yin, cols_v[b * 7 + 2][dsl], cols_v[b * 7 + 5][dsl])
            xf = (x + 1.0) * 0.5 * 255.0
            yf = (y + 1.0) * 0.5 * 255.0
            zf = (z + 1.0) * 0.5 * 255.0
            xi = jnp.minimum(jnp.maximum(xf.astype(jnp.int32), 0), 254)
            yi = jnp.minimum(jnp.maximum(yf.astype(jnp.int32), 0), 254)
            zi = jnp.minimum(jnp.maximum(zf.astype(jnp.int32), 0), 254)
            fi = f.astype(jnp.int32) * _DHW
            i00 = fi + zi * (_H * _W) + yi * _W + xi
            for k in range(4):
                idx_v[b * 4 + k][dsl] = i00 + _ZY_OFFS[k]
            w_v[b][0, dsl] = xf - xi.astype(jnp.float32)
            w_v[b][1, dsl] = yf - yi.astype(jnp.float32)
            w_v[b][2, dsl] = zf - zi.astype(jnp.float32)
            return carry

        lax.fori_loop(0, _CHUNK // 16, compute_idx, 0)
        for k in range(4):
            pltpu.async_copy(vol_hbm.at[idx_v[b * 4 + k]], val_v[b * 4 + k],
                             sems[b])

    def finish(c, b):
        for k in range(4):
            pltpu.make_async_copy(vol_hbm.at[idx_v[b * 4 + k]],
                                  val_v[b * 4 + k], sems[b]).wait()

        def compute_out(j, carry):
            dsl = pl.ds(j * 16, 16)
            wx1 = w_v[b][0, dsl]
            wy1 = w_v[b][1, dsl]
            wz1 = w_v[b][2, dsl]
            wx0 = 1.0 - wx1
            wy0 = 1.0 - wy1
            wz0 = 1.0 - wz1

            def pair(k):
                packed = val_v[b * 4 + k][dsl]
                v0 = lax.bitcast_convert_type(packed << 16, jnp.float32)
                v1 = lax.bitcast_convert_type(packed & jnp.int32(-65536),
                                              jnp.float32)
                return v0 * wx0 + v1 * wx1

            acc = pair(0) * (wy0 * wz0)
            acc = acc + pair(1) * (wy1 * wz0)
            acc = acc + pair(2) * (wy0 * wz1)
            acc = acc + pair(3) * (wy1 * wz1)
            out_v[b][dsl] = acc
            return carry

        lax.fori_loop(0, _CHUNK // 16, compute_out, 0)
        base = base_w + c * _CHUNK
        pltpu.sync_copy(out_v[b], out_hbm.at[pl.ds(base, _CHUNK)])

    prep(0, 0)

    def loop_body(i, carry):
        c = 2 * i
        prep(c + 1, 1)
        finish(c, 0)

        @pl.when(i < _NCHUNK // 2 - 1)
        def _():
            prep(c + 2, 0)

        finish(c + 1, 1)
        return carry

    lax.fori_loop(0, _NCHUNK // 2, loop_body, 0)


_sc_call = functools.partial(
    pl.kernel,
    out_type=jax.ShapeDtypeStruct((_N,), jnp.float32),
    mesh=plsc.VectorSubcoreMesh(core_axis_name="c", subcore_axis_name="s"),
    scratch_types=[
        [pltpu.VMEM((_CHUNK,), jnp.float32) for _ in range(14)],
        [pltpu.VMEM((_CHUNK,), jnp.int32) for _ in range(8)],
        [pltpu.VMEM((3, _CHUNK), jnp.float32) for _ in range(2)],
        [pltpu.VMEM((_CHUNK,), jnp.int32) for _ in range(8)],
        [pltpu.VMEM((_CHUNK,), jnp.float32) for _ in range(2)],
        [pltpu.SemaphoreType.DMA for _ in range(2)],
    ],
)(_tec_body)


@jax.jit
def kernel(norm_samples, alpha_volume_yin, alpha_volume_yang):
    cols = [norm_samples[:, q] for q in range(7)]
    flat = jnp.concatenate(
        [alpha_volume_yin.reshape(-1), alpha_volume_yang.reshape(-1)])
    lo = flat.astype(jnp.bfloat16)
    hi = jnp.concatenate([flat[1:], flat[:1]]).astype(jnp.bfloat16)
    packed = jax.lax.bitcast_convert_type(
        jnp.stack([lo, hi], axis=-1), jnp.int32)
    return _sc_call(*cols, packed)


# speedup vs baseline: 3.5683x; 3.5683x over previous
import functools

import jax
import jax.numpy as jnp
from jax import lax
from jax.experimental import pallas as pl
from jax.experimental.pallas import tpu as pltpu
from jax.experimental.pallas import tpu_sc as plsc

_D = _H = _W = 256
_N = 1048576
_DHW = _D * _H * _W

_NC = 2
_NS = 16
_NW = _NC * _NS
_PER_W = _N // _NW
_CHUNK = 2048
_NCHUNK = _PER_W // _CHUNK

_ZY_OFFS = (0, _W, _H * _W, _H * _W + _W)


def _tec_body(c0, c1, c2, c3, c4, c5, c6, vol_hbm, out_hbm,
              cols_v, idx_v, w_v, val_v, out_v, sems):
    cols_hbm = (c0, c1, c2, c3, c4, c5, c6)
    wid = lax.axis_index("s") * _NC + lax.axis_index("c")
    base_w = wid * _PER_W

    def prep(c, b):
        base = base_w + c * _CHUNK
        for q in range(7):
            pltpu.sync_copy(cols_hbm[q].at[pl.ds(base, _CHUNK)],
                            cols_v[b * 7 + q])

        def compute_idx(j, carry):
            dsl = pl.ds(j * 16, 16)
            f = cols_v[b * 7 + 6][dsl]
            yin = f == 0.0
            x = jnp.where(yin, cols_v[b * 7 + 0][dsl], cols_v[b * 7 + 3][dsl])
            y = jnp.where(yin, cols_v[b * 7 + 1][dsl], cols_v[b * 7 + 4][dsl])
            z = jnp.where(yin, cols_v[b * 7 + 2][dsl], cols_v[b * 7 + 5][dsl])
            xf = (x + 1.0) * 0.5 * 255.0
            yf = (y + 1.0) * 0.5 * 255.0
            zf = (z + 1.0) * 0.5 * 255.0
            xi = jnp.minimum(jnp.maximum(xf.astype(jnp.int32), 0), 254)
            yi = jnp.minimum(jnp.maximum(yf.astype(jnp.int32), 0), 254)
            zi = jnp.minimum(jnp.maximum(zf.astype(jnp.int32), 0), 254)
            fi = f.astype(jnp.int32) * _DHW
            i00 = fi + zi * (_H * _W) + yi * _W + xi
            for k in range(4):
                idx_v[b * 4 + k][dsl] = i00 + _ZY_OFFS[k]
            w_v[b][0, dsl] = xf - xi.astype(jnp.float32)
            w_v[b][1, dsl] = yf - yi.astype(jnp.float32)
            w_v[b][2, dsl] = zf - zi.astype(jnp.float32)
            return carry

        lax.fori_loop(0, _CHUNK // 16, compute_idx, 0)
        for k in range(4):
            pltpu.async_copy(vol_hbm.at[idx_v[b * 4 + k]], val_v[b * 4 + k],
                             sems[b])

    def finish(c, b):
        for k in range(4):
            pltpu.make_async_copy(vol_hbm.at[idx_v[b * 4 + k]],
                                  val_v[b * 4 + k], sems[b]).wait()

        def compute_out(j, carry):
            dsl = pl.ds(j * 16, 16)
            wx1 = w_v[b][0, dsl]
            wy1 = w_v[b][1, dsl]
            wz1 = w_v[b][2, dsl]
            wx0 = 1.0 - wx1
            wy0 = 1.0 - wy1
            wz0 = 1.0 - wz1

            def pair(k):
                packed = val_v[b * 4 + k][dsl]
                v0 = lax.bitcast_convert_type(packed << 16, jnp.float32)
                v1 = lax.bitcast_convert_type(packed & jnp.int32(-65536),
                                              jnp.float32)
                return v0 * wx0 + v1 * wx1

            acc = pair(0) * (wy0 * wz0)
            acc = acc + pair(1) * (wy1 * wz0)
            acc = acc + pair(2) * (wy0 * wz1)
            acc = acc + pair(3) * (wy1 * wz1)
            out_v[b][dsl] = acc
            return carry

        lax.fori_loop(0, _CHUNK // 16, compute_out, 0)
        base = base_w + c * _CHUNK
        pltpu.sync_copy(out_v[b], out_hbm.at[pl.ds(base, _CHUNK)])

    prep(0, 0)

    def loop_body(i, carry):
        c = 2 * i
        prep(c + 1, 1)
        finish(c, 0)

        @pl.when(i < _NCHUNK // 2 - 1)
        def _():
            prep(c + 2, 0)

        finish(c + 1, 1)
        return carry

    lax.fori_loop(0, _NCHUNK // 2, loop_body, 0)


_sc_call = functools.partial(
    pl.kernel,
    out_type=jax.ShapeDtypeStruct((_N,), jnp.float32),
    mesh=plsc.VectorSubcoreMesh(core_axis_name="c", subcore_axis_name="s"),
    scratch_types=[
        [pltpu.VMEM((_CHUNK,), jnp.float32) for _ in range(14)],
        [pltpu.VMEM((_CHUNK,), jnp.int32) for _ in range(8)],
        [pltpu.VMEM((3, _CHUNK), jnp.float32) for _ in range(2)],
        [pltpu.VMEM((_CHUNK,), jnp.int32) for _ in range(8)],
        [pltpu.VMEM((_CHUNK,), jnp.float32) for _ in range(2)],
        [pltpu.SemaphoreType.DMA for _ in range(2)],
    ],
)(_tec_body)


@jax.jit
def kernel(norm_samples, alpha_volume_yin, alpha_volume_yang):
    cols = [norm_samples[:, q] for q in range(7)]
    flat = jnp.concatenate(
        [alpha_volume_yin.reshape(-1), alpha_volume_yang.reshape(-1)])
    fb = jax.lax.bitcast_convert_type(flat, jnp.int32)
    r = (fb + 0x7FFF + ((fb >> 16) & 1)) >> 16
    rs = jnp.concatenate([r[1:], r[:1]])
    packed = (r & 0xFFFF) | (rs << 16)
    return _sc_call(*cols, packed)


# speedup vs baseline: 3.6592x; 1.0255x over previous
import functools

import jax
import jax.numpy as jnp
from jax import lax
from jax.experimental import pallas as pl
from jax.experimental.pallas import tpu as pltpu
from jax.experimental.pallas import tpu_sc as plsc

_D = _H = _W = 256
_N = 1048576
_DHW = _D * _H * _W

_NC = 2
_NS = 16
_NW = _NC * _NS
_PER_W = _N // _NW
_CHUNK = 2048
_NCHUNK = _PER_W // _CHUNK

_ZY_OFFS = (0, _W, _H * _W, _H * _W + _W)


def _tec_body(c0, c1, c2, c3, c4, c5, c6, vol_hbm, out_hbm,
              cols_v, idx_v, w_v, val_v, out_v, sems):
    cols_hbm = (c0, c1, c2, c3, c4, c5, c6)
    wid = lax.axis_index("s") * _NC + lax.axis_index("c")
    base_w = wid * _PER_W

    def prep(c, b):
        base = base_w + c * _CHUNK
        for q in range(7):
            pltpu.sync_copy(cols_hbm[q].at[pl.ds(base, _CHUNK)],
                            cols_v[b * 7 + q])

        def compute_idx(j, carry):
            dsl = pl.ds(j * 16, 16)
            f = cols_v[b * 7 + 6][dsl]
            yin = f == 0.0
            x = jnp.where(yin, cols_v[b * 7 + 0][dsl], cols_v[b * 7 + 3][dsl])
            y = jnp.where(yin, cols_v[b * 7 + 1][dsl], cols_v[b * 7 + 4][dsl])
            z = jnp.where(yin, cols_v[b * 7 + 2][dsl], cols_v[b * 7 + 5][dsl])
            xf = (x + 1.0) * 0.5 * 255.0
            yf = (y + 1.0) * 0.5 * 255.0
            zf = (z + 1.0) * 0.5 * 255.0
            xi = jnp.minimum(jnp.maximum(xf.astype(jnp.int32), 0), 254)
            yi = jnp.minimum(jnp.maximum(yf.astype(jnp.int32), 0), 254)
            zi = jnp.minimum(jnp.maximum(zf.astype(jnp.int32), 0), 254)
            fi = f.astype(jnp.int32) * _DHW
            i00 = fi + zi * (_H * _W) + yi * _W + xi
            for k in range(4):
                idx_v[b * 4 + k][dsl] = i00 + _ZY_OFFS[k]
            w_v[b][0, dsl] = xf - xi.astype(jnp.float32)
            w_v[b][1, dsl] = yf - yi.astype(jnp.float32)
            w_v[b][2, dsl] = zf - zi.astype(jnp.float32)
            return carry

        lax.fori_loop(0, _CHUNK // 16, compute_idx, 0)
        for k in range(4):
            pltpu.async_copy(vol_hbm.at[idx_v[b * 4 + k]], val_v[b * 4 + k],
                             sems[b])

    def finish(c, b):
        for k in range(4):
            pltpu.make_async_copy(vol_hbm.at[idx_v[b * 4 + k]],
                                  val_v[b * 4 + k], sems[b]).wait()

        def compute_out(j, carry):
            dsl = pl.ds(j * 16, 16)
            wx1 = w_v[b][0, dsl]
            wy1 = w_v[b][1, dsl]
            wz1 = w_v[b][2, dsl]
            wx0 = 1.0 - wx1
            wy0 = 1.0 - wy1
            wz0 = 1.0 - wz1

            def pair(k):
                packed = val_v[b * 4 + k][dsl]
                v0 = lax.bitcast_convert_type(packed << 16, jnp.float32)
                v1 = lax.bitcast_convert_type(packed & jnp.int32(-65536),
                                              jnp.float32)
                return v0 * wx0 + v1 * wx1

            acc = pair(0) * (wy0 * wz0)
            acc = acc + pair(1) * (wy1 * wz0)
            acc = acc + pair(2) * (wy0 * wz1)
            acc = acc + pair(3) * (wy1 * wz1)
            out_v[b][dsl] = acc
            return carry

        lax.fori_loop(0, _CHUNK // 16, compute_out, 0)
        base = base_w + c * _CHUNK
        pltpu.sync_copy(out_v[b], out_hbm.at[pl.ds(base, _CHUNK)])

    prep(0, 0)

    def loop_body(i, carry):
        c = 2 * i
        prep(c + 1, 1)
        finish(c, 0)

        @pl.when(i < _NCHUNK // 2 - 1)
        def _():
            prep(c + 2, 0)

        finish(c + 1, 1)
        return carry

    lax.fori_loop(0, _NCHUNK // 2, loop_body, 0)


_sc_call = functools.partial(
    pl.kernel,
    out_type=jax.ShapeDtypeStruct((_N,), jnp.float32),
    mesh=plsc.VectorSubcoreMesh(core_axis_name="c", subcore_axis_name="s"),
    scratch_types=[
        [pltpu.VMEM((_CHUNK,), jnp.float32) for _ in range(14)],
        [pltpu.VMEM((_CHUNK,), jnp.int32) for _ in range(8)],
        [pltpu.VMEM((3, _CHUNK), jnp.float32) for _ in range(2)],
        [pltpu.VMEM((_CHUNK,), jnp.int32) for _ in range(8)],
        [pltpu.VMEM((_CHUNK,), jnp.float32) for _ in range(2)],
        [pltpu.SemaphoreType.DMA for _ in range(2)],
    ],
)(_tec_body)


@jax.jit
def kernel(norm_samples, alpha_volume_yin, alpha_volume_yang):
    cols = [norm_samples[:, q] for q in range(7)]
    flat = jnp.concatenate(
        [alpha_volume_yin.reshape(-1), alpha_volume_yang.reshape(-1)])
    fb = jax.lax.bitcast_convert_type(flat, jnp.int32)
    fbs = jax.lax.bitcast_convert_type(
        jnp.concatenate([flat[1:], flat[:1]]), jnp.int32)

    def rne(v):
        return (v + 0x7FFF + ((v >> 16) & 1)) >> 16

    packed = (rne(fb) & 0xFFFF) | (rne(fbs) << 16)
    return _sc_call(*cols, packed)
